# baseline (speedup 1.0000x reference)
import functools

import jax
import jax.numpy as jnp
from jax import lax
from jax.experimental import pallas as pl
from jax.experimental.pallas import tpu as pltpu
from jax.experimental.pallas import tpu_sc as plsc

B = 16384
EDIM = 64
ALPHA = 0.001
BETA = 0.001
L2RG = 0.0001

NC = 2
NS = 16
NW = NC * NS
RPW = B // NW
CHUNK = 128
NCH = RPW // CHUNK
NGC = CHUNK // 16

_sc_mesh = plsc.VectorSubcoreMesh(core_axis_name="c", subcore_axis_name="s")


@functools.partial(
    pl.kernel,
    out_type=[jax.ShapeDtypeStruct((B,), jnp.float32) for _ in range(5)]
    + [jax.ShapeDtypeStruct((NW, 48), jnp.float32),
       jax.ShapeDtypeStruct((CHUNK, EDIM), jnp.float32)],
    mesh=_sc_mesh,
    compiler_params=pltpu.CompilerParams(needs_layout_passes=False,
                                         use_tc_tiling_on_sc=False),
    scratch_types=[
        pltpu.VMEM((RPW,), jnp.int32),
        pltpu.VMEM((RPW,), jnp.int32),
        pltpu.VMEM((CHUNK, EDIM), jnp.float32),
        pltpu.VMEM((CHUNK, EDIM), jnp.float32),
        pltpu.VMEM((RPW,), jnp.float32),
        pltpu.VMEM((RPW,), jnp.float32),
        pltpu.VMEM((RPW,), jnp.float32),
        pltpu.VMEM((RPW,), jnp.float32),
        pltpu.VMEM((RPW,), jnp.float32),
        pltpu.VMEM((48,), jnp.float32),
        pltpu.VMEM((EDIM,), jnp.float32),
        pltpu.VMEM((EDIM,), jnp.float32),
        pltpu.SemaphoreType.DMA,
        pltpu.SemaphoreType.DMA,
    ],
)
def _sc_main(u_idx, nu_idx, p_idx, n_idx, ut, it, uw_hbm, iw_hbm,
             pu_out, nu_out, pi_out, ni_out, dot_out, sq_out, dummy_out,
             idx_sa, idx_sb, rows_a, rows_b,
             pu_v, nu_v, pi_v, ni_v, dot_v, sq_v,
             uw_s, iw_s, sem_a, sem_b):
    wid = lax.axis_index("s") * NC + lax.axis_index("c")
    base = wid * RPW

    pltpu.sync_copy(uw_hbm, uw_s)
    pltpu.sync_copy(iw_hbm, iw_s)

    def load_idx(idx_hbm, idx_s):
        pltpu.sync_copy(idx_hbm.at[pl.ds(base, RPW)], idx_s)

    def fire_rows(tab, idx_s, rows, sem, c):
        def _fire(je, _):
            rv = idx_s[pl.ds(c * CHUNK + je * 16, 16)]
            for dd in range(16):
                r = rv[dd]
                pltpu.async_copy(tab.at[pl.ds(r, 1)],
                                 rows.at[pl.ds(je * 16 + dd, 1)], sem)
            return 0
        lax.fori_loop(0, CHUNK // 16, _fire, 0)

    def drain_rows(rows, sem):
        pltpu.make_async_copy(dummy_out, rows, sem).wait()

    iota = lax.iota(jnp.int32, 16)
    zero = jnp.zeros((16,), jnp.float32)

    load_idx(u_idx, idx_sa)
    load_idx(p_idx, idx_sb)
    su_acc = zero
    sp_acc = zero
    for c in range(NCH):
        fire_rows(ut, idx_sa, rows_a, sem_a, c)
        fire_rows(it, idx_sb, rows_b, sem_b, c)
        drain_rows(rows_a, sem_a)
        drain_rows(rows_b, sem_b)

        def _group_up(g, carry, c=c):
            su, sp = carry
            r0 = g * 16
            jv = iota + r0

            pu_a = pi_a = dot_a = su_g = sp_g = zero
            for k in range(EDIM // 16):
                uwk = uw_s[pl.ds(k * 16, 16)]
                iwk = iw_s[pl.ds(k * 16, 16)]
                for dd in range(16):
                    d = k * 16 + dd
                    dv = jnp.full((16,), d, jnp.int32)
                    cu = plsc.load_gather(rows_a, [jv, dv])
                    cp = plsc.load_gather(rows_b, [jv, dv])
                    pu_a = pu_a + cu * uwk[dd]
                    pi_a = pi_a + cp * iwk[dd]
                    dot_a = dot_a + cu * cp
                    su_g = su_g + cu * cu
                    sp_g = sp_g + cp * cp
            pu_v[pl.ds(c * CHUNK + r0, 16)] = pu_a
            pi_v[pl.ds(c * CHUNK + r0, 16)] = pi_a
            dot_v[pl.ds(c * CHUNK + r0, 16)] = dot_a
            return su + su_g, sp + sp_g

        su_acc, sp_acc = lax.fori_loop(0, NGC, _group_up, (su_acc, sp_acc))

    load_idx(nu_idx, idx_sa)
    for c in range(NCH):
        fire_rows(ut, idx_sa, rows_a, sem_a, c)
        drain_rows(rows_a, sem_a)

        def _group_nu(g, _, c=c):
            r0 = g * 16
            jv = iota + r0

            acc = zero
            for k in range(EDIM // 16):
                uwk = uw_s[pl.ds(k * 16, 16)]
                for dd in range(16):
                    d = k * 16 + dd
                    dv = jnp.full((16,), d, jnp.int32)
                    cn = plsc.load_gather(rows_a, [jv, dv])
                    acc = acc + cn * uwk[dd]
            nu_v[pl.ds(c * CHUNK + r0, 16)] = acc
            return 0

        lax.fori_loop(0, NGC, _group_nu, 0)

    load_idx(n_idx, idx_sb)
    sn_acc = zero
    for c in range(NCH):
        fire_rows(it, idx_sb, rows_b, sem_b, c)
        drain_rows(rows_b, sem_b)

        def _group_ni(g, carry, c=c):
            r0 = g * 16
            jv = iota + r0

            ni_a = sn_g = zero
            for k in range(EDIM // 16):
                iwk = iw_s[pl.ds(k * 16, 16)]
                for dd in range(16):
                    d = k * 16 + dd
                    dv = jnp.full((16,), d, jnp.int32)
                    cn = plsc.load_gather(rows_b, [jv, dv])
                    ni_a = ni_a + cn * iwk[dd]
                    sn_g = sn_g + cn * cn
            ni_v[pl.ds(c * CHUNK + r0, 16)] = ni_a
            return carry + sn_g

        sn_acc = lax.fori_loop(0, NGC, _group_ni, sn_acc)

    sq_v[pl.ds(0, 16)] = su_acc
    sq_v[pl.ds(16, 16)] = sp_acc
    sq_v[pl.ds(32, 16)] = sn_acc

    pltpu.sync_copy(pu_v, pu_out.at[pl.ds(base, RPW)])
    pltpu.sync_copy(nu_v, nu_out.at[pl.ds(base, RPW)])
    pltpu.sync_copy(pi_v, pi_out.at[pl.ds(base, RPW)])
    pltpu.sync_copy(ni_v, ni_out.at[pl.ds(base, RPW)])
    pltpu.sync_copy(dot_v, dot_out.at[pl.ds(base, RPW)])
    pltpu.sync_copy(sq_v, sq_out.at[wid])


def _tc_loss_body(pu_ref, nu_ref, pi_ref, ni_ref, dot_ref, rate_ref, sq_ref,
                  ub_ref, ib_ref, out_ref):
    ub = ub_ref[0, 0]
    ib = ib_ref[0, 0]
    pu = pu_ref[...] + ub
    nu = nu_ref[...] + ub
    pi = pi_ref[...] + ib
    ni = ni_ref[...] + ib
    dot = dot_ref[...]

    pred = 1.0 + 4.0 * jax.nn.sigmoid(jax.nn.sigmoid(pu) * jax.nn.sigmoid(pi) * dot)
    rate_loss = jnp.mean((pred - rate_ref[...]) ** 2)
    user_loss = jnp.mean(jax.nn.softplus(-pu)) + jnp.mean(jax.nn.softplus(nu))
    item_loss = jnp.mean(jax.nn.softplus(-pi)) + jnp.mean(jax.nn.softplus(ni))
    reg = jnp.sum(sq_ref[...]) * (0.5 / B)
    loss = rate_loss + ALPHA * user_loss + BETA * item_loss + L2RG * reg
    out_ref[...] = loss.reshape(1, 1)


_tc_loss = pl.pallas_call(
    _tc_loss_body,
    out_shape=jax.ShapeDtypeStruct((1, 1), jnp.float32),
)


def kernel(user, u_ir, nbr, item, rate, neg_user, neg_item,
           user_table, item_table, user_w, user_b, item_w, item_b):
    del u_ir, nbr
    pu, nu, pi, ni, dot, sq, _ = _sc_main(
        user.astype(jnp.int32), neg_user.astype(jnp.int32),
        item.astype(jnp.int32), neg_item.astype(jnp.int32),
        user_table, item_table, user_w.reshape(EDIM), item_w.reshape(EDIM))
    shp = (128, 128)
    loss = _tc_loss(pu.reshape(shp), nu.reshape(shp), pi.reshape(shp),
                    ni.reshape(shp), dot.reshape(shp), rate.reshape(shp),
                    sq, user_b.reshape(1, 1), item_b.reshape(1, 1))
    return loss.reshape(())


# speedup vs baseline: 1.5442x; 1.5442x over previous
import functools

import jax
import jax.numpy as jnp
from jax import lax
from jax.experimental import pallas as pl
from jax.experimental.pallas import tpu as pltpu
from jax.experimental.pallas import tpu_sc as plsc

B = 16384
EDIM = 64
ALPHA = 0.001
BETA = 0.001
L2RG = 0.0001

NC = 2
NS = 16
NW = NC * NS
RPW = B // NW
CHUNK = 128
NCH = RPW // CHUNK

_sc_mesh = plsc.VectorSubcoreMesh(core_axis_name="c", subcore_axis_name="s")


@functools.partial(
    pl.kernel,
    out_type=[jax.ShapeDtypeStruct((B, EDIM), jnp.float32) for _ in range(2)]
    + [jax.ShapeDtypeStruct((CHUNK, EDIM), jnp.float32)],
    mesh=_sc_mesh,
    scratch_types=[
        pltpu.VMEM((RPW,), jnp.int32),
        pltpu.VMEM((CHUNK, EDIM), jnp.float32),
        pltpu.SemaphoreType.DMA,
    ],
)
def _sc_gather2(idx1, idx2, tab, out1, out2, dummy_out, idx_v, rows, sem):
    wid = lax.axis_index("s") * NC + lax.axis_index("c")
    base = wid * RPW

    def fire_rows(c):
        def _fire(je, _):
            rv = idx_v[pl.ds(c * CHUNK + je * 16, 16)]
            for dd in range(16):
                r = rv[dd]
                pltpu.async_copy(tab.at[pl.ds(r, 1)],
                                 rows.at[pl.ds(je * 16 + dd, 1)], sem)
            return 0
        lax.fori_loop(0, CHUNK // 16, _fire, 0)

    for idx_hbm, out in ((idx1, out1), (idx2, out2)):
        pltpu.sync_copy(idx_hbm.at[pl.ds(base, RPW)], idx_v)
        for c in range(NCH):
            fire_rows(c)
            pltpu.make_async_copy(dummy_out, rows, sem).wait()
            pltpu.sync_copy(rows, out.at[pl.ds(base + c * CHUNK, CHUNK)])


def _tc_loss_body(u_ref, nu_ref, p_ref, n_ref, rate_ref,
                  uw_ref, ub_ref, iw_ref, ib_ref, out_ref):
    u = u_ref[...]
    nu = nu_ref[...]
    p = p_ref[...]
    n = n_ref[...]
    uw = uw_ref[...].reshape(1, EDIM)
    iw = iw_ref[...].reshape(1, EDIM)
    ub = ub_ref[0, 0]
    ib = ib_ref[0, 0]

    pu = jnp.sum(u * uw, axis=1, keepdims=True) + ub
    nu_l = jnp.sum(nu * uw, axis=1, keepdims=True) + ub
    pi = jnp.sum(p * iw, axis=1, keepdims=True) + ib
    ni = jnp.sum(n * iw, axis=1, keepdims=True) + ib
    dot = jnp.sum(u * p, axis=1, keepdims=True)

    pred = 1.0 + 4.0 * jax.nn.sigmoid(jax.nn.sigmoid(pu) * jax.nn.sigmoid(pi) * dot)
    rate_loss = jnp.mean((pred - rate_ref[...]) ** 2)
    user_loss = jnp.mean(jax.nn.softplus(-pu)) + jnp.mean(jax.nn.softplus(nu_l))
    item_loss = jnp.mean(jax.nn.softplus(-pi)) + jnp.mean(jax.nn.softplus(ni))
    reg = (jnp.sum(u * u) + jnp.sum(p * p) + jnp.sum(n * n)) * (0.5 / B)
    loss = rate_loss + ALPHA * user_loss + BETA * item_loss + L2RG * reg
    out_ref[...] = loss.reshape(1, 1)


_tc_loss = pl.pallas_call(
    _tc_loss_body,
    out_shape=jax.ShapeDtypeStruct((1, 1), jnp.float32),
)


def kernel(user, u_ir, nbr, item, rate, neg_user, neg_item,
           user_table, item_table, user_w, user_b, item_w, item_b):
    del u_ir, nbr
    u_emb, nu_emb = _sc_gather2(user.astype(jnp.int32),
                                neg_user.astype(jnp.int32), user_table)[:2]
    p_emb, n_emb = _sc_gather2(item.astype(jnp.int32),
                               neg_item.astype(jnp.int32), item_table)[:2]
    loss = _tc_loss(u_emb, nu_emb, p_emb, n_emb, rate.reshape(B, 1),
                    user_w, user_b.reshape(1, 1), item_w, item_b.reshape(1, 1))
    return loss.reshape(())


# speedup vs baseline: 1.5531x; 1.0058x over previous
import functools

import jax
import jax.numpy as jnp
from jax import lax
from jax.experimental import pallas as pl
from jax.experimental.pallas import tpu as pltpu
from jax.experimental.pallas import tpu_sc as plsc

B = 16384
EDIM = 64
ALPHA = 0.001
BETA = 0.001
L2RG = 0.0001

NC = 2
NS = 16
NW = NC * NS
RPW = B // NW
CHUNK = 256
NCH = RPW // CHUNK

_sc_mesh = plsc.VectorSubcoreMesh(core_axis_name="c", subcore_axis_name="s")


@functools.partial(
    pl.kernel,
    out_type=[jax.ShapeDtypeStruct((B, EDIM), jnp.float32) for _ in range(2)]
    + [jax.ShapeDtypeStruct((CHUNK, EDIM), jnp.float32)],
    mesh=_sc_mesh,
    scratch_types=[
        pltpu.VMEM((RPW,), jnp.int32),
        pltpu.VMEM((CHUNK, EDIM), jnp.float32),
        pltpu.SemaphoreType.DMA,
    ],
)
def _sc_gather2(idx1, idx2, tab, out1, out2, dummy_out, idx_v, rows, sem):
    wid = lax.axis_index("s") * NC + lax.axis_index("c")
    base = wid * RPW

    def fire_rows(c):
        def _fire(je, _):
            rv = idx_v[pl.ds(c * CHUNK + je * 16, 16)]
            for dd in range(16):
                r = rv[dd]
                pltpu.async_copy(tab.at[pl.ds(r, 1)],
                                 rows.at[pl.ds(je * 16 + dd, 1)], sem)
            return 0
        lax.fori_loop(0, CHUNK // 16, _fire, 0)

    for idx_hbm, out in ((idx1, out1), (idx2, out2)):
        pltpu.sync_copy(idx_hbm.at[pl.ds(base, RPW)], idx_v)
        for c in range(NCH):
            fire_rows(c)
            pltpu.make_async_copy(dummy_out, rows, sem).wait()
            pltpu.sync_copy(rows, out.at[pl.ds(base + c * CHUNK, CHUNK)])


def _tc_loss_body(u_ref, nu_ref, p_ref, n_ref, rate_ref,
                  uw_ref, ub_ref, iw_ref, ib_ref, out_ref):
    u = u_ref[...]
    nu = nu_ref[...]
    p = p_ref[...]
    n = n_ref[...]
    uw = uw_ref[...].reshape(1, EDIM)
    iw = iw_ref[...].reshape(1, EDIM)
    ub = ub_ref[0, 0]
    ib = ib_ref[0, 0]

    pu = jnp.sum(u * uw, axis=1, keepdims=True) + ub
    nu_l = jnp.sum(nu * uw, axis=1, keepdims=True) + ub
    pi = jnp.sum(p * iw, axis=1, keepdims=True) + ib
    ni = jnp.sum(n * iw, axis=1, keepdims=True) + ib
    dot = jnp.sum(u * p, axis=1, keepdims=True)

    pred = 1.0 + 4.0 * jax.nn.sigmoid(jax.nn.sigmoid(pu) * jax.nn.sigmoid(pi) * dot)
    rate_loss = jnp.mean((pred - rate_ref[...]) ** 2)
    user_loss = jnp.mean(jax.nn.softplus(-pu)) + jnp.mean(jax.nn.softplus(nu_l))
    item_loss = jnp.mean(jax.nn.softplus(-pi)) + jnp.mean(jax.nn.softplus(ni))
    reg = (jnp.sum(u * u) + jnp.sum(p * p) + jnp.sum(n * n)) * (0.5 / B)
    loss = rate_loss + ALPHA * user_loss + BETA * item_loss + L2RG * reg
    out_ref[...] = loss.reshape(1, 1)


_tc_loss = pl.pallas_call(
    _tc_loss_body,
    out_shape=jax.ShapeDtypeStruct((1, 1), jnp.float32),
)


def kernel(user, u_ir, nbr, item, rate, neg_user, neg_item,
           user_table, item_table, user_w, user_b, item_w, item_b):
    del u_ir, nbr
    u_emb, nu_emb = _sc_gather2(user.astype(jnp.int32),
                                neg_user.astype(jnp.int32), user_table)[:2]
    p_emb, n_emb = _sc_gather2(item.astype(jnp.int32),
                               neg_item.astype(jnp.int32), item_table)[:2]
    loss = _tc_loss(u_emb, nu_emb, p_emb, n_emb, rate.reshape(B, 1),
                    user_w, user_b.reshape(1, 1), item_w, item_b.reshape(1, 1))
    return loss.reshape(())


# speedup vs baseline: 1.5564x; 1.0022x over previous
import functools

import jax
import jax.numpy as jnp
from jax import lax
from jax.experimental import pallas as pl
from jax.experimental.pallas import tpu as pltpu
from jax.experimental.pallas import tpu_sc as plsc

B = 16384
EDIM = 64
ALPHA = 0.001
BETA = 0.001
L2RG = 0.0001

NC = 2
NS = 16
NW = NC * NS
RPW = B // NW
CHUNK = 512
NCH = RPW // CHUNK

_sc_mesh = plsc.VectorSubcoreMesh(core_axis_name="c", subcore_axis_name="s")


@functools.partial(
    pl.kernel,
    out_type=[jax.ShapeDtypeStruct((B, EDIM), jnp.float32) for _ in range(2)]
    + [jax.ShapeDtypeStruct((CHUNK, EDIM), jnp.float32)],
    mesh=_sc_mesh,
    scratch_types=[
        pltpu.VMEM((RPW,), jnp.int32),
        pltpu.VMEM((CHUNK, EDIM), jnp.float32),
        pltpu.SemaphoreType.DMA,
    ],
)
def _sc_gather2(idx1, idx2, tab, out1, out2, dummy_out, idx_v, rows, sem):
    wid = lax.axis_index("s") * NC + lax.axis_index("c")
    base = wid * RPW

    def fire_rows(c):
        def _fire(je, _):
            rv = idx_v[pl.ds(c * CHUNK + je * 16, 16)]
            for dd in range(16):
                r = rv[dd]
                pltpu.async_copy(tab.at[pl.ds(r, 1)],
                                 rows.at[pl.ds(je * 16 + dd, 1)], sem)
            return 0
        lax.fori_loop(0, CHUNK // 16, _fire, 0)

    for idx_hbm, out in ((idx1, out1), (idx2, out2)):
        pltpu.sync_copy(idx_hbm.at[pl.ds(base, RPW)], idx_v)
        for c in range(NCH):
            fire_rows(c)
            pltpu.make_async_copy(dummy_out, rows, sem).wait()
            pltpu.sync_copy(rows, out.at[pl.ds(base + c * CHUNK, CHUNK)])


def _tc_loss_body(u_ref, nu_ref, p_ref, n_ref, rate_ref,
                  uw_ref, ub_ref, iw_ref, ib_ref, out_ref):
    u = u_ref[...]
    nu = nu_ref[...]
    p = p_ref[...]
    n = n_ref[...]
    uw = uw_ref[...].reshape(1, EDIM)
    iw = iw_ref[...].reshape(1, EDIM)
    ub = ub_ref[0, 0]
    ib = ib_ref[0, 0]

    pu = jnp.sum(u * uw, axis=1, keepdims=True) + ub
    nu_l = jnp.sum(nu * uw, axis=1, keepdims=True) + ub
    pi = jnp.sum(p * iw, axis=1, keepdims=True) + ib
    ni = jnp.sum(n * iw, axis=1, keepdims=True) + ib
    dot = jnp.sum(u * p, axis=1, keepdims=True)

    pred = 1.0 + 4.0 * jax.nn.sigmoid(jax.nn.sigmoid(pu) * jax.nn.sigmoid(pi) * dot)
    rate_loss = jnp.mean((pred - rate_ref[...]) ** 2)
    user_loss = jnp.mean(jax.nn.softplus(-pu)) + jnp.mean(jax.nn.softplus(nu_l))
    item_loss = jnp.mean(jax.nn.softplus(-pi)) + jnp.mean(jax.nn.softplus(ni))
    reg = (jnp.sum(u * u) + jnp.sum(p * p) + jnp.sum(n * n)) * (0.5 / B)
    loss = rate_loss + ALPHA * user_loss + BETA * item_loss + L2RG * reg
    out_ref[...] = loss.reshape(1, 1)


_tc_loss = pl.pallas_call(
    _tc_loss_body,
    out_shape=jax.ShapeDtypeStruct((1, 1), jnp.float32),
)


def kernel(user, u_ir, nbr, item, rate, neg_user, neg_item,
           user_table, item_table, user_w, user_b, item_w, item_b):
    del u_ir, nbr
    u_emb, nu_emb = _sc_gather2(user.astype(jnp.int32),
                                neg_user.astype(jnp.int32), user_table)[:2]
    p_emb, n_emb = _sc_gather2(item.astype(jnp.int32),
                               neg_item.astype(jnp.int32), item_table)[:2]
    loss = _tc_loss(u_emb, nu_emb, p_emb, n_emb, rate.reshape(B, 1),
                    user_w, user_b.reshape(1, 1), item_w, item_b.reshape(1, 1))
    return loss.reshape(())
